# baseline (speedup 1.0000x reference)
import jax
import jax.numpy as jnp
from jax import lax
from jax.experimental import pallas as pl
from jax.experimental.pallas import tpu as pltpu
from jax.experimental.pallas import tpu_sc as plsc

N = 100000
NP = 100096
NPQ = NP // 8
E = 1600000
EP = 1605632
H = 32
HH = 16
NSC = 2
NTILES = 16
CHUNK_E = 448
INNER = 8
SUPER_E = CHUNK_E * INNER
E_PER_TILE = EP // NTILES
SCHUNKS = E_PER_TILE // SUPER_E
NSLOT = 3
ROWS_PER_TILE = NP // NTILES
ZCHUNK = 368
BP = 544

_f32 = jnp.float32


def _bd(w):
    return jnp.kron(jnp.eye(8, dtype=_f32), w)


def _packb(b):
    return jnp.tile(b, 8).reshape(1, 128)


def _encmm_body(f_ref, we0, we1, be0, be1, wm00, wm01, wm10, wm11,
                h0_ref, h1_ref, g0_ref, g1_ref):
    f = f_ref[...]
    h0 = jnp.maximum(jnp.dot(f, we0[...], preferred_element_type=_f32)
                     + be0[...], 0.0)
    h1 = jnp.maximum(jnp.dot(f, we1[...], preferred_element_type=_f32)
                     + be1[...], 0.0)
    h0_ref[...] = h0
    h1_ref[...] = h1
    g0_ref[...] = (jnp.dot(h0, wm00[...], preferred_element_type=_f32)
                   + jnp.dot(h1, wm10[...], preferred_element_type=_f32))
    g1_ref[...] = (jnp.dot(h0, wm01[...], preferred_element_type=_f32)
                   + jnp.dot(h1, wm11[...], preferred_element_type=_f32))


_WSPEC = pl.BlockSpec((128, 128), lambda i: (0, 0))
_BSPEC = pl.BlockSpec((1, 128), lambda i: (0, 0))
_RSPEC = pl.BlockSpec((BP, 128), lambda i: (i, 0))

_enc_mm = pl.pallas_call(
    _encmm_body,
    grid=(NPQ // BP,),
    in_specs=[_RSPEC] + [_WSPEC] * 2 + [_BSPEC] * 2 + [_WSPEC] * 4,
    out_specs=[_RSPEC] * 4,
    out_shape=[jax.ShapeDtypeStruct((NPQ, 128), _f32)] * 4,
)


def _enc_body(f_ref, we0, we1, be0, be1, h0_ref, h1_ref):
    f = f_ref[...]
    h0_ref[...] = jnp.maximum(
        jnp.dot(f, we0[...], preferred_element_type=_f32) + be0[...], 0.0)
    h1_ref[...] = jnp.maximum(
        jnp.dot(f, we1[...], preferred_element_type=_f32) + be1[...], 0.0)


_enc = pl.pallas_call(
    _enc_body,
    grid=(NPQ // BP,),
    in_specs=[_RSPEC] + [_WSPEC] * 2 + [_BSPEC] * 2,
    out_specs=[_RSPEC] * 2,
    out_shape=[jax.ShapeDtypeStruct((NPQ, 128), _f32)] * 2,
)


def _updmm_body(a0, a1, h0, h1, bu0, bu1, wm00, wm01, wm10, wm11,
                hn0_ref, hn1_ref, g0_ref, g1_ref):
    hn0 = jnp.maximum(h0[...] + jnp.maximum(a0[...] + bu0[...], 0.0), 0.0)
    hn1 = jnp.maximum(h1[...] + jnp.maximum(a1[...] + bu1[...], 0.0), 0.0)
    hn0_ref[...] = hn0
    hn1_ref[...] = hn1
    g0_ref[...] = (jnp.dot(hn0, wm00[...], preferred_element_type=_f32)
                   + jnp.dot(hn1, wm10[...], preferred_element_type=_f32))
    g1_ref[...] = (jnp.dot(hn0, wm01[...], preferred_element_type=_f32)
                   + jnp.dot(hn1, wm11[...], preferred_element_type=_f32))


_upd_mm = pl.pallas_call(
    _updmm_body,
    grid=(NPQ // BP,),
    in_specs=[_RSPEC] * 4 + [_BSPEC] * 2 + [_WSPEC] * 4,
    out_specs=[_RSPEC] * 4,
    out_shape=[jax.ShapeDtypeStruct((NPQ, 128), _f32)] * 4,
)


def _upd_body(a0, a1, h0, h1, bu0, bu1, hn0_ref, hn1_ref):
    hn0_ref[...] = jnp.maximum(
        h0[...] + jnp.maximum(a0[...] + bu0[...], 0.0), 0.0)
    hn1_ref[...] = jnp.maximum(
        h1[...] + jnp.maximum(a1[...] + bu1[...], 0.0), 0.0)


_upd = pl.pallas_call(
    _upd_body,
    grid=(NPQ // BP,),
    in_specs=[_RSPEC] * 4 + [_BSPEC] * 2,
    out_specs=[_RSPEC] * 2,
    out_shape=[jax.ShapeDtypeStruct((NPQ, 128), _f32)] * 2,
)


def _seg_body(g0, g1, s1d, d1d, o0, o1,
              acc, idxs, idxd, rows_a, rows_b, rows_c,
              sem_z, sem_ga, sem_gb, sem_gc, sem_sa, sem_sb, sem_sc):
    c = lax.axis_index("c")
    t = lax.axis_index("s")
    zero = jnp.zeros((HH,), _f32)
    rows = (rows_a, rows_b, rows_c)
    sem_g = (sem_ga, sem_gb, sem_gc)
    sem_s = (sem_sa, sem_sb, sem_sc)

    zsrc = rows_a.at[pl.ds(0, ZCHUNK)]

    def _zb(i, carry):
        rows_a[i, :] = zero
        return carry
    lax.fori_loop(0, ZCHUNK, _zb, 0, unroll=4)
    zd = [
        pltpu.async_copy(
            zsrc, acc.at[pl.ds(t * ROWS_PER_TILE + q * ZCHUNK, ZCHUNK)],
            sem_z)
        for q in range(ROWS_PER_TILE // ZCHUNK)
    ]
    for d in zd:
        d.wait()
    plsc.subcore_barrier()

    def _edges(gref):
        def _schunk(ci, carry):
            eb = t * E_PER_TILE + ci * SUPER_E
            pltpu.sync_copy(s1d.at[pl.ds(eb, SUPER_E)], idxs)
            pltpu.sync_copy(d1d.at[pl.ds(eb, SUPER_E)], idxd)
            gd = [None] * INNER
            sd = [None] * INNER

            def _gather(j):
                s = j % NSLOT
                return pltpu.async_copy(
                    gref.at[idxs.at[pl.ds(j * CHUNK_E, CHUNK_E)]],
                    rows[s], sem_g[s])

            gd[0] = _gather(0)
            gd[1] = _gather(1)
            for j in range(INNER):
                s = j % NSLOT
                gd[j].wait()
                if j + 2 < INNER:
                    if j >= 1:
                        sd[j - 1].wait()
                    gd[j + 2] = _gather(j + 2)
                sd[j] = pltpu.async_copy(
                    rows[s], acc.at[idxd.at[pl.ds(j * CHUNK_E, CHUNK_E)]],
                    sem_s[s], add=True)
            sd[INNER - 3].wait()
            sd[INNER - 2].wait()
            sd[INNER - 1].wait()
            return carry
        lax.fori_loop(0, SCHUNKS, _schunk, 0)

    @pl.when(c == 0)
    def _():
        _edges(g0)

    @pl.when(c == 1)
    def _():
        _edges(g1)

    plsc.subcore_barrier()

    r0 = t * ROWS_PER_TILE

    @pl.when(c == 0)
    def _():
        pltpu.sync_copy(acc.at[pl.ds(r0, ROWS_PER_TILE)],
                        o0.at[pl.ds(r0, ROWS_PER_TILE)])

    @pl.when(c == 1)
    def _():
        pltpu.sync_copy(acc.at[pl.ds(r0, ROWS_PER_TILE)],
                        o1.at[pl.ds(r0, ROWS_PER_TILE)])


_seg = pl.kernel(
    _seg_body,
    out_type=[jax.ShapeDtypeStruct((NP, HH), _f32)] * 2,
    mesh=plsc.VectorSubcoreMesh(core_axis_name="c", subcore_axis_name="s",
                                num_cores=NSC, num_subcores=NTILES),
    scratch_types=[
        pltpu.VMEM_SHARED((NP, HH), _f32),
        pltpu.VMEM((SUPER_E,), jnp.int32),
        pltpu.VMEM((SUPER_E,), jnp.int32),
        pltpu.VMEM((CHUNK_E, HH), _f32),
        pltpu.VMEM((CHUNK_E, HH), _f32),
        pltpu.VMEM((CHUNK_E, HH), _f32),
        pltpu.SemaphoreType.DMA,
        pltpu.SemaphoreType.DMA,
        pltpu.SemaphoreType.DMA,
        pltpu.SemaphoreType.DMA,
        pltpu.SemaphoreType.DMA,
        pltpu.SemaphoreType.DMA,
        pltpu.SemaphoreType.DMA,
    ],
    compiler_params=pltpu.CompilerParams(use_tc_tiling_on_sc=False),
)


def _unflat(x):
    return x.reshape(NP, HH)


def _flat(x):
    return x.reshape(NPQ, 128)


def kernel(item_feat, pattern_feat, edge_index, W_item, b_item, W_pat, b_pat,
           W_i2p, b_i2p, W_p2i, b_p2i):
    i_idx = edge_index[0].astype(jnp.int32)
    p_idx = edge_index[1].astype(jnp.int32)
    pad = N + (jnp.arange(EP - E, dtype=jnp.int32) % 64)
    i1d = jnp.concatenate([i_idx, pad])
    p1d = jnp.concatenate([p_idx, pad])

    fiP = _flat(jnp.pad(item_feat, ((0, NP - N), (0, 0))))
    fpP = _flat(jnp.pad(pattern_feat, ((0, NP - N), (0, 0))))

    wi_e0, wi_e1 = _bd(W_item[:, :HH]), _bd(W_item[:, HH:])
    wp_e0, wp_e1 = _bd(W_pat[:, :HH]), _bd(W_pat[:, HH:])
    bi_e0, bi_e1 = _packb(b_item[:HH]), _packb(b_item[HH:])
    bp_e0, bp_e1 = _packb(b_pat[:HH]), _packb(b_pat[HH:])
    wi2p = [[_bd(W_i2p[r * HH:(r + 1) * HH, c * HH:(c + 1) * HH])
             for c in range(2)] for r in range(2)]
    wp2i = [[_bd(W_p2i[r * HH:(r + 1) * HH, c * HH:(c + 1) * HH])
             for c in range(2)] for r in range(2)]
    bi2p = [_packb(b_i2p[:HH]), _packb(b_i2p[HH:])]
    bp2i = [_packb(b_p2i[:HH]), _packb(b_p2i[HH:])]

    hi0, hi1, g0, g1 = _enc_mm(fiP, wi_e0, wi_e1, bi_e0, bi_e1,
                               wi2p[0][0], wi2p[0][1], wi2p[1][0], wi2p[1][1])
    hp0, hp1 = _enc(fpP, wp_e0, wp_e1, bp_e0, bp_e1)

    a0, a1 = _seg(_unflat(g0), _unflat(g1), i1d, p1d)
    hp0, hp1, g0, g1 = _upd_mm(_flat(a0), _flat(a1), hp0, hp1,
                               bi2p[0], bi2p[1],
                               wp2i[0][0], wp2i[0][1], wp2i[1][0], wp2i[1][1])
    a0, a1 = _seg(_unflat(g0), _unflat(g1), p1d, i1d)
    hi0, hi1, g0, g1 = _upd_mm(_flat(a0), _flat(a1), hi0, hi1,
                               bp2i[0], bp2i[1],
                               wi2p[0][0], wi2p[0][1], wi2p[1][0], wi2p[1][1])
    a0, a1 = _seg(_unflat(g0), _unflat(g1), i1d, p1d)
    hp0, hp1, g0, g1 = _upd_mm(_flat(a0), _flat(a1), hp0, hp1,
                               bi2p[0], bi2p[1],
                               wp2i[0][0], wp2i[0][1], wp2i[1][0], wp2i[1][1])
    a0, a1 = _seg(_unflat(g0), _unflat(g1), p1d, i1d)
    hi0, hi1 = _upd(_flat(a0), _flat(a1), hi0, hi1, bp2i[0], bp2i[1])

    def _unpack(x0, x1):
        a = _unflat(x0)[:N]
        b = _unflat(x1)[:N]
        return jnp.concatenate([a, b], axis=1)

    return (_unpack(hi0, hi1), _unpack(hp0, hp1))
